# Initial kernel scaffold; baseline (speedup 1.0000x reference)
import jax
import jax.numpy as jnp
from jax.experimental import pallas as pl


def kernel(input_ids, word_table, special_table):
    raise NotImplementedError("write your pallas kernel here")


# speedup vs baseline: 1.7883x; 1.7883x over previous
import functools

import jax
import jax.numpy as jnp
from jax import lax
from jax.experimental import pallas as pl
from jax.experimental.pallas import tpu as pltpu
from jax.experimental.pallas import tpu_sc as plsc

D = 64
BOS_ID = 1
EOS_ID = 2

NC = 2
NS = 16
NW = NC * NS

CHUNK = 1024
IDXW = 128
NSTREAM = CHUNK // IDXW
VPC = CHUNK // 16


@functools.lru_cache(maxsize=None)
def _make_kernel(n):
    assert n % (NW * CHUNK) == 0
    per_worker = n // NW
    chunks_per_worker = per_worker // CHUNK
    mesh = plsc.VectorSubcoreMesh(core_axis_name="c", subcore_axis_name="s")

    @functools.partial(
        pl.kernel,
        mesh=mesh,
        compiler_params=pltpu.CompilerParams(
            use_tc_tiling_on_sc=False, needs_layout_passes=False),
        out_type=jax.ShapeDtypeStruct((n, D), jnp.float32),
        scratch_types=[
            pltpu.VMEM((NSTREAM, IDXW), jnp.int32),
            pltpu.VMEM((CHUNK, D), jnp.float32),
            pltpu.VMEM((2, D), jnp.float32),
            pltpu.SemaphoreType.DMA,
        ],
    )
    def k(idx_hbm, table_hbm, spec_hbm, out_hbm, idx_v, rows_v, spec_v, sem):
        wid = lax.axis_index("s") * NC + lax.axis_index("c")
        row0 = wid * (per_worker // IDXW)
        base = wid * per_worker
        pltpu.sync_copy(spec_hbm, spec_v)

        def chunk_body(g, carry):
            pltpu.sync_copy(idx_hbm.at[pl.ds(row0 + g * NSTREAM, NSTREAM)],
                            idx_v)
            waits = [
                pltpu.async_copy(
                    table_hbm.at[idx_v.at[j]],
                    rows_v.at[pl.ds(j * IDXW, IDXW)],
                    sem,
                )
                for j in range(NSTREAM)
            ]
            for w in waits:
                w.wait()

            def patch_body(t, c2):
                j = t // (IDXW // 16)
                c = (t % (IDXW // 16)) * 16
                ids = idx_v[j, pl.ds(c, 16)]
                hit = ((ids == BOS_ID) | (ids == EOS_ID)).astype(jnp.int32)
                cnt = jnp.sum(hit)

                @pl.when(cnt > 0)
                def _():
                    for lane in range(16):
                        sid = ids[lane]

                        @pl.when((sid == BOS_ID) | (sid == EOS_ID))
                        def _():
                            srow = sid - BOS_ID
                            row = t * 16 + lane
                            for gcol in range(D // 16):
                                rows_v[row, pl.ds(gcol * 16, 16)] = (
                                    spec_v[srow, pl.ds(gcol * 16, 16)])

                return c2

            lax.fori_loop(0, VPC, patch_body, 0)

            pltpu.sync_copy(rows_v, out_hbm.at[pl.ds(base + g * CHUNK, CHUNK)])
            return carry

        lax.fori_loop(0, chunks_per_worker, chunk_body, 0)

    return k


def kernel(input_ids, word_table, special_table):
    b, t = input_ids.shape
    n = b * t
    idx = input_ids.astype(jnp.int32).reshape(n // IDXW, IDXW)
    out = _make_kernel(n)(idx, word_table, special_table)
    return out.reshape(b, t, D)


# speedup vs baseline: 1.8750x; 1.0485x over previous
import functools

import jax
import jax.numpy as jnp
from jax import lax
from jax.experimental import pallas as pl
from jax.experimental.pallas import tpu as pltpu
from jax.experimental.pallas import tpu_sc as plsc

D = 64
BOS_ID = 1
EOS_ID = 2

NC = 2
NS = 16
NW = NC * NS

NBUF = 3
CHUNK = 512
IDXW = 128
NSTREAM = CHUNK // IDXW
VPC = CHUNK // 16


@functools.lru_cache(maxsize=None)
def _make_kernel(n):
    assert n % (NW * CHUNK) == 0
    per_worker = n // NW
    nchunks = per_worker // CHUNK
    nrows = per_worker // IDXW
    assert nchunks >= NBUF
    mesh = plsc.VectorSubcoreMesh(core_axis_name="c", subcore_axis_name="s")

    @functools.partial(
        pl.kernel,
        mesh=mesh,
        compiler_params=pltpu.CompilerParams(
            use_tc_tiling_on_sc=False, needs_layout_passes=False),
        out_type=jax.ShapeDtypeStruct((n, D), jnp.float32),
        scratch_types=[
            pltpu.VMEM((nrows, IDXW), jnp.int32),
            pltpu.VMEM((NBUF, CHUNK, D), jnp.float32),
            pltpu.VMEM((2, D), jnp.float32),
            pltpu.SemaphoreType.DMA((NBUF,)),
            pltpu.SemaphoreType.DMA((NBUF,)),
        ],
    )
    def k(idx_hbm, table_hbm, spec_hbm, out_hbm, idx_v, rows_v, spec_v,
          gsem, osem):
        wid = lax.axis_index("s") * NC + lax.axis_index("c")
        base = wid * per_worker
        pltpu.sync_copy(spec_hbm, spec_v)
        pltpu.sync_copy(idx_hbm.at[pl.ds(wid * nrows, nrows)], idx_v)

        def fire_gathers(q):
            qb = q % NBUF
            for j in range(NSTREAM):
                pltpu.async_copy(
                    table_hbm.at[idx_v.at[q * NSTREAM + j]],
                    rows_v.at[qb].at[pl.ds(j * IDXW, IDXW)],
                    gsem.at[qb],
                )

        def wait_gathers(buf):
            for j in range(NSTREAM):
                pltpu.make_async_copy(
                    rows_v.at[buf].at[pl.ds(j * IDXW, IDXW)],
                    out_hbm.at[pl.ds(base, IDXW)],
                    gsem.at[buf],
                ).wait()

        def wait_out(buf):
            pltpu.make_async_copy(
                rows_v.at[buf], out_hbm.at[pl.ds(base, CHUNK)], osem.at[buf]
            ).wait()

        fire_gathers(0)
        fire_gathers(1)

        def chunk_body(g, carry):
            buf = g % NBUF
            wait_gathers(buf)

            def patch_body(t, c2):
                jrow = g * NSTREAM + t // (IDXW // 16)
                c = (t % (IDXW // 16)) * 16
                ids = idx_v[jrow, pl.ds(c, 16)]
                hit = ((ids == BOS_ID) | (ids == EOS_ID)).astype(jnp.int32)
                cnt = jnp.sum(hit)

                @pl.when(cnt > 0)
                def _():
                    for lane in range(16):
                        sid = ids[lane]

                        @pl.when((sid == BOS_ID) | (sid == EOS_ID))
                        def _():
                            srow = sid - BOS_ID
                            row = t * 16 + lane
                            for gcol in range(D // 16):
                                rows_v[buf, row, pl.ds(gcol * 16, 16)] = (
                                    spec_v[srow, pl.ds(gcol * 16, 16)])

                return c2

            lax.fori_loop(0, VPC, patch_body, 0)

            pltpu.async_copy(
                rows_v.at[buf], out_hbm.at[pl.ds(base + g * CHUNK, CHUNK)],
                osem.at[buf])

            @pl.when(g + 2 < nchunks)
            def _():
                @pl.when(g >= 1)
                def _():
                    wait_out((g + 2) % NBUF)

                fire_gathers(g + 2)

            return carry

        lax.fori_loop(0, nchunks, chunk_body, 0)
        for r in range(NBUF):
            wait_out((nchunks - 1 - r) % NBUF)

    return k


def kernel(input_ids, word_table, special_table):
    b, t = input_ids.shape
    n = b * t
    idx = input_ids.astype(jnp.int32).reshape(n // IDXW, IDXW)
    out = _make_kernel(n)(idx, word_table, special_table)
    return out.reshape(b, t, D)


# speedup vs baseline: 2.1295x; 1.1357x over previous
import functools

import jax
import jax.numpy as jnp
from jax import lax
from jax.experimental import pallas as pl
from jax.experimental.pallas import tpu as pltpu
from jax.experimental.pallas import tpu_sc as plsc

D = 64
BOS_ID = 1
EOS_ID = 2

NC = 2
NS = 16
NW = NC * NS

BLK = 128
GBUF = 3
TBUF = 2
PITCH = 129


@functools.lru_cache(maxsize=None)
def _make_kernel(b, t):
    nbb = b // BLK
    bb_per_w = nbb // NW
    nunits = bb_per_w * t
    assert nunits >= GBUF
    mesh = plsc.VectorSubcoreMesh(core_axis_name="c", subcore_axis_name="s")

    @functools.partial(
        pl.kernel,
        mesh=mesh,
        compiler_params=pltpu.CompilerParams(
            use_tc_tiling_on_sc=False, needs_layout_passes=False),
        out_type=jax.ShapeDtypeStruct((t * (D // 8) * nbb, 8, BLK),
                                      jnp.float32),
        scratch_types=[
            pltpu.VMEM((t, bb_per_w * BLK), jnp.int32),
            pltpu.VMEM((GBUF, BLK, D), jnp.float32),
            pltpu.VMEM((TBUF, D, PITCH), jnp.float32),
            pltpu.VMEM((2, D), jnp.float32),
            pltpu.SemaphoreType.DMA((GBUF,)),
            pltpu.SemaphoreType.DMA((TBUF,)),
        ],
    )
    def k(ids_hbm, table_hbm, spec_hbm, out_hbm, ids_v, gbuf, tbuf, spec_v,
          gsem, osem):
        wid = lax.axis_index("s") * NC + lax.axis_index("c")
        bb0 = wid * bb_per_w
        pltpu.sync_copy(spec_hbm, spec_v)
        pltpu.sync_copy(ids_hbm.at[:, pl.ds(bb0 * BLK, bb_per_w * BLK)],
                        ids_v)

        riota = lax.iota(jnp.int32, 16)
        rows_dg = [riota + dg * 16 for dg in range(4)]

        def unit_tb(u):
            return u // bb_per_w, u % bb_per_w

        def fire_gather(u):
            ut, ub = unit_tb(u)
            pltpu.async_copy(
                table_hbm.at[ids_v.at[ut].at[pl.ds(ub * BLK, BLK)]],
                gbuf.at[u % GBUF],
                gsem.at[u % GBUF],
            )

        def wait_gather(u):
            pltpu.make_async_copy(
                table_hbm.at[pl.ds(0, BLK)], gbuf.at[u % GBUF],
                gsem.at[u % GBUF]
            ).wait()

        def wait_outs(slot):
            for dblk in range(D // 8):
                pltpu.make_async_copy(
                    tbuf.at[slot].at[pl.ds(dblk * 8, 8), pl.ds(0, BLK)],
                    out_hbm.at[0],
                    osem.at[slot],
                ).wait()

        fire_gather(0)
        fire_gather(1)

        def unit_body(u, carry):
            gslot = u % GBUF
            tslot = u % TBUF
            ut, ub = unit_tb(u)
            wait_gather(u)

            def patch_body(v, c2):
                ids = ids_v[ut, pl.ds(ub * BLK + v * 16, 16)]
                hit = ((ids == BOS_ID) | (ids == EOS_ID)).astype(jnp.int32)
                cnt = jnp.sum(hit)

                @pl.when(cnt > 0)
                def _():
                    for lane in range(16):
                        sid = ids[lane]

                        @pl.when((sid == BOS_ID) | (sid == EOS_ID))
                        def _():
                            srow = sid - BOS_ID
                            row = v * 16 + lane
                            for dg in range(4):
                                gbuf[gslot, row, pl.ds(dg * 16, 16)] = (
                                    spec_v[srow, pl.ds(dg * 16, 16)])

                return c2

            lax.fori_loop(0, BLK // 16, patch_body, 0)

            @pl.when(u + 2 < nunits)
            def _():
                fire_gather(u + 2)

            @pl.when(u >= TBUF)
            def _():
                wait_outs(tslot)

            def tr_body(tok, c2):
                tokv = jnp.full((16,), tok, jnp.int32)
                for dg in range(4):
                    val = gbuf[gslot, tok, pl.ds(dg * 16, 16)]
                    plsc.store_scatter(tbuf.at[tslot], [rows_dg[dg], tokv],
                                       val)
                return c2

            lax.fori_loop(0, BLK, tr_body, 0)

            for dblk in range(D // 8):
                pltpu.async_copy(
                    tbuf.at[tslot].at[pl.ds(dblk * 8, 8), pl.ds(0, BLK)],
                    out_hbm.at[(ut * (D // 8) + dblk) * nbb + bb0 + ub],
                    osem.at[tslot],
                )

            return carry

        lax.fori_loop(0, nunits, unit_body, 0)
        wait_outs((nunits - 2) % TBUF)
        wait_outs((nunits - 1) % TBUF)

    return k


def kernel(input_ids, word_table, special_table):
    b, t = input_ids.shape
    ids_t = input_ids.astype(jnp.int32).T
    out = _make_kernel(b, t)(ids_t, word_table, special_table)
    out = out.reshape(t, D // 8, b // BLK, 8, BLK)
    return out.transpose(2, 4, 0, 1, 3).reshape(b, t, D)


# speedup vs baseline: 2.1866x; 1.0268x over previous
import functools

import jax
import jax.numpy as jnp
from jax import lax
from jax.experimental import pallas as pl
from jax.experimental.pallas import tpu as pltpu
from jax.experimental.pallas import tpu_sc as plsc

D = 64
BOS_ID = 1
EOS_ID = 2

NC = 2
NS = 16
NW = NC * NS

BLK = 128
GBUF = 6
TBUF = 2
PITCH = 129


@functools.lru_cache(maxsize=None)
def _make_kernel(b, t):
    nbb = b // BLK
    bb_per_w = nbb // NW
    nunits = bb_per_w * t
    assert nunits >= GBUF
    mesh = plsc.VectorSubcoreMesh(core_axis_name="c", subcore_axis_name="s")

    @functools.partial(
        pl.kernel,
        mesh=mesh,
        compiler_params=pltpu.CompilerParams(
            use_tc_tiling_on_sc=False, needs_layout_passes=False),
        out_type=jax.ShapeDtypeStruct((t * (D // 8) * nbb, 8, BLK),
                                      jnp.float32),
        scratch_types=[
            pltpu.VMEM((t, bb_per_w * BLK), jnp.int32),
            pltpu.VMEM((GBUF, BLK, D), jnp.float32),
            pltpu.VMEM((TBUF, D, PITCH), jnp.float32),
            pltpu.VMEM((2, D), jnp.float32),
            pltpu.SemaphoreType.DMA((GBUF,)),
            pltpu.SemaphoreType.DMA((TBUF,)),
        ],
    )
    def k(ids_hbm, table_hbm, spec_hbm, out_hbm, ids_v, gbuf, tbuf, spec_v,
          gsem, osem):
        wid = lax.axis_index("s") * NC + lax.axis_index("c")
        bb0 = wid * bb_per_w
        pltpu.sync_copy(spec_hbm, spec_v)
        pltpu.sync_copy(ids_hbm.at[:, pl.ds(bb0 * BLK, bb_per_w * BLK)],
                        ids_v)

        riota = lax.iota(jnp.int32, 16)
        rows_dg = [riota + dg * 16 for dg in range(4)]

        def unit_tb(u):
            return u // bb_per_w, u % bb_per_w

        def fire_gather(u):
            ut, ub = unit_tb(u)
            pltpu.async_copy(
                table_hbm.at[ids_v.at[ut].at[pl.ds(ub * BLK, BLK)]],
                gbuf.at[u % GBUF],
                gsem.at[u % GBUF],
            )

        def wait_gather(u):
            pltpu.make_async_copy(
                table_hbm.at[pl.ds(0, BLK)], gbuf.at[u % GBUF],
                gsem.at[u % GBUF]
            ).wait()

        def wait_outs(slot):
            for dblk in range(D // 8):
                pltpu.make_async_copy(
                    tbuf.at[slot].at[pl.ds(dblk * 8, 8), pl.ds(0, BLK)],
                    out_hbm.at[0],
                    osem.at[slot],
                ).wait()

        for p in range(GBUF - 1):
            fire_gather(p)

        def unit_body(u, carry):
            gslot = u % GBUF
            tslot = u % TBUF
            ut, ub = unit_tb(u)
            wait_gather(u)

            def patch_body(v, c2):
                ids = ids_v[ut, pl.ds(ub * BLK + v * 16, 16)]
                hit = (ids == BOS_ID) | (ids == EOS_ID)
                cnt = plsc.all_reduce_population_count(hit)[0]

                @pl.when(cnt > 0)
                def _():
                    for lane in range(16):
                        sid = ids[lane]

                        @pl.when((sid == BOS_ID) | (sid == EOS_ID))
                        def _():
                            srow = sid - BOS_ID
                            row = v * 16 + lane
                            for dg in range(4):
                                gbuf[gslot, row, pl.ds(dg * 16, 16)] = (
                                    spec_v[srow, pl.ds(dg * 16, 16)])

                return c2

            lax.fori_loop(0, BLK // 16, patch_body, 0, unroll=4)

            @pl.when(u + GBUF - 1 < nunits)
            def _():
                fire_gather(u + GBUF - 1)

            @pl.when(u >= TBUF)
            def _():
                wait_outs(tslot)

            def tr_body(tok, c2):
                tokv = jnp.full((16,), tok, jnp.int32)
                for dg in range(4):
                    val = gbuf[gslot, tok, pl.ds(dg * 16, 16)]
                    plsc.store_scatter(tbuf.at[tslot], [rows_dg[dg], tokv],
                                       val)
                return c2

            lax.fori_loop(0, BLK, tr_body, 0, unroll=8)

            for dblk in range(D // 8):
                pltpu.async_copy(
                    tbuf.at[tslot].at[pl.ds(dblk * 8, 8), pl.ds(0, BLK)],
                    out_hbm.at[(ut * (D // 8) + dblk) * nbb + bb0 + ub],
                    osem.at[tslot],
                )

            return carry

        lax.fori_loop(0, nunits, unit_body, 0)
        wait_outs((nunits - 2) % TBUF)
        wait_outs((nunits - 1) % TBUF)

    return k


def kernel(input_ids, word_table, special_table):
    b, t = input_ids.shape
    ids_t = input_ids.astype(jnp.int32).T
    out = _make_kernel(b, t)(ids_t, word_table, special_table)
    out = out.reshape(t, D // 8, b // BLK, 8, BLK)
    return out.transpose(2, 4, 0, 1, 3).reshape(b, t, D)


# speedup vs baseline: 2.1957x; 1.0042x over previous
import functools

import jax
import jax.numpy as jnp
from jax import lax
from jax.experimental import pallas as pl
from jax.experimental.pallas import tpu as pltpu
from jax.experimental.pallas import tpu_sc as plsc

D = 64
BOS_ID = 1
EOS_ID = 2

NC = 2
NS = 16
NW = NC * NS

BLK = 128
GBUF = 6
TBUF = 2
PITCH = 129


@functools.lru_cache(maxsize=None)
def _make_kernel(b, t):
    nbb = b // BLK
    bb_per_w = nbb // NW
    nunits = bb_per_w * t
    assert nunits >= GBUF
    mesh = plsc.VectorSubcoreMesh(core_axis_name="c", subcore_axis_name="s")

    @functools.partial(
        pl.kernel,
        mesh=mesh,
        compiler_params=pltpu.CompilerParams(
            use_tc_tiling_on_sc=False, needs_layout_passes=False),
        out_type=jax.ShapeDtypeStruct((t, D // 8, nbb, 8, BLK),
                                      jnp.float32),
        scratch_types=[
            pltpu.VMEM((t, bb_per_w * BLK), jnp.int32),
            pltpu.VMEM((GBUF, BLK, D), jnp.float32),
            pltpu.VMEM((TBUF, D // 8, 8, PITCH), jnp.float32),
            pltpu.VMEM((2, D), jnp.float32),
            pltpu.SemaphoreType.DMA((GBUF,)),
            pltpu.SemaphoreType.DMA((TBUF,)),
        ],
    )
    def k(ids_hbm, table_hbm, spec_hbm, out_hbm, ids_v, gbuf, tbuf, spec_v,
          gsem, osem):
        wid = lax.axis_index("s") * NC + lax.axis_index("c")
        bb0 = wid * bb_per_w
        pltpu.sync_copy(spec_hbm, spec_v)
        pltpu.sync_copy(ids_hbm.at[:, pl.ds(bb0 * BLK, bb_per_w * BLK)],
                        ids_v)

        riota = lax.iota(jnp.int32, 16)
        dblk_dg = [(riota + dg * 16) >> 3 for dg in range(4)]
        dsub_dg = [(riota + dg * 16) & 7 for dg in range(4)]

        def unit_tb(u):
            return u // bb_per_w, u % bb_per_w

        def fire_gather(u):
            ut, ub = unit_tb(u)
            pltpu.async_copy(
                table_hbm.at[ids_v.at[ut].at[pl.ds(ub * BLK, BLK)]],
                gbuf.at[u % GBUF],
                gsem.at[u % GBUF],
            )

        def wait_gather(u):
            pltpu.make_async_copy(
                table_hbm.at[pl.ds(0, BLK)], gbuf.at[u % GBUF],
                gsem.at[u % GBUF]
            ).wait()

        def wait_outs(slot):
            pltpu.make_async_copy(
                tbuf.at[slot, :, :, pl.ds(0, BLK)],
                out_hbm.at[0, :, 0],
                osem.at[slot],
            ).wait()

        for p in range(GBUF - 1):
            fire_gather(p)

        def unit_body(u, carry):
            gslot = u % GBUF
            tslot = u % TBUF
            ut, ub = unit_tb(u)
            wait_gather(u)

            def patch_body(v, c2):
                ids = ids_v[ut, pl.ds(ub * BLK + v * 16, 16)]
                hit = (ids == BOS_ID) | (ids == EOS_ID)
                cnt = plsc.all_reduce_population_count(hit)[0]

                @pl.when(cnt > 0)
                def _():
                    for lane in range(16):
                        sid = ids[lane]

                        @pl.when((sid == BOS_ID) | (sid == EOS_ID))
                        def _():
                            srow = sid - BOS_ID
                            row = v * 16 + lane
                            for dg in range(4):
                                gbuf[gslot, row, pl.ds(dg * 16, 16)] = (
                                    spec_v[srow, pl.ds(dg * 16, 16)])

                return c2

            lax.fori_loop(0, BLK // 16, patch_body, 0, unroll=4)

            @pl.when(u + GBUF - 1 < nunits)
            def _():
                fire_gather(u + GBUF - 1)

            @pl.when(u >= TBUF)
            def _():
                wait_outs(tslot)

            def tr_body(tok, c2):
                tokv = jnp.full((16,), tok, jnp.int32)
                for dg in range(4):
                    val = gbuf[gslot, tok, pl.ds(dg * 16, 16)]
                    plsc.store_scatter(
                        tbuf.at[tslot],
                        [dblk_dg[dg], dsub_dg[dg], tokv], val)
                return c2

            lax.fori_loop(0, BLK, tr_body, 0, unroll=8)

            pltpu.async_copy(
                tbuf.at[tslot, :, :, pl.ds(0, BLK)],
                out_hbm.at[ut, :, bb0 + ub],
                osem.at[tslot],
            )

            return carry

        lax.fori_loop(0, nunits, unit_body, 0)
        wait_outs((nunits - 2) % TBUF)
        wait_outs((nunits - 1) % TBUF)

    return k


def kernel(input_ids, word_table, special_table):
    b, t = input_ids.shape
    ids_t = input_ids.astype(jnp.int32).T
    out = _make_kernel(b, t)(ids_t, word_table, special_table)
    out = out.reshape(t, D // 8, b // BLK, 8, BLK)
    return out.transpose(2, 4, 0, 1, 3).reshape(b, t, D)
